# baseline (speedup 1.0000x reference)
import jax
import jax.numpy as jnp
from jax import lax
from jax.experimental import pallas as pl
from jax.experimental.pallas import tpu as pltpu
from jax.experimental.pallas import tpu_sc as plsc

B = 16384
K = 16
PACK = 8
NC = 2
NS = 16
NW = NC * NS
BPW = B // NW
NCHUNK = 4
CHUNK = BPW // NCHUNK
NBUF = 2


def _sc_kernel(w_hbm, h_hbm, uidx_hbm, iidx_hbm, out_hbm,
               uidx_v, iidx_v, usup_v, isup_v, urows_v, vrows_v, out_v, sems):
    wid = lax.axis_index("c") * NS + lax.axis_index("s")

    pltpu.sync_copy(uidx_hbm.at[wid], uidx_v)
    pltpu.sync_copy(iidx_hbm.at[wid], iidx_v)

    for c in range(NCHUNK):
        for t in range(CHUNK // 16):
            sl = pl.ds(t * 16, 16)
            usup_v[c, sl] = uidx_v[c, sl] >> 3
            isup_v[c, sl] = iidx_v[c, sl] >> 3

    def issue(c):
        buf = c % NBUF
        dst = pl.ds(buf * CHUNK, CHUNK)
        return (
            pltpu.async_copy(w_hbm.at[usup_v.at[c]], urows_v.at[dst], sems.at[buf]),
            pltpu.async_copy(h_hbm.at[isup_v.at[c]], vrows_v.at[dst], sems.at[buf]),
        )

    lane = lax.iota(jnp.int32, 16)
    pending = {0: issue(0), 1: issue(1)}

    for c in range(NCHUNK):
        cu, cv = pending.pop(c)
        cu.wait()
        cv.wait()
        buf = c % NBUF
        for g in range(CHUNK // 16):
            sl = pl.ds(g * 16, 16)
            rows = buf * CHUNK + g * 16 + lane
            ucol0 = (uidx_v[c, sl] & 7) << 4
            vcol0 = (iidx_v[c, sl] & 7) << 4
            acc = jnp.zeros((16,), jnp.float32)
            for k in range(K):
                u = plsc.load_gather(urows_v, [rows, ucol0 + k])
                v = plsc.load_gather(vrows_v, [rows, vcol0 + k])
                acc = acc + u * v
            out_v[pl.ds(c * CHUNK + g * 16, 16)] = acc
        if c + NBUF < NCHUNK:
            pending[c + NBUF] = issue(c + NBUF)

    pltpu.sync_copy(out_v, out_hbm.at[pl.ds(wid * BPW, BPW)])


@jax.jit
def _run(w, h, uidx, iidx):
    mesh = plsc.VectorSubcoreMesh(core_axis_name="c", subcore_axis_name="s")
    fn = pl.kernel(
        _sc_kernel,
        mesh=mesh,
        compiler_params=pltpu.CompilerParams(needs_layout_passes=False),
        out_type=jax.ShapeDtypeStruct((B,), jnp.float32),
        scratch_types=[
            pltpu.VMEM((NCHUNK, CHUNK), jnp.int32),
            pltpu.VMEM((NCHUNK, CHUNK), jnp.int32),
            pltpu.VMEM((NCHUNK, CHUNK), jnp.int32),
            pltpu.VMEM((NCHUNK, CHUNK), jnp.int32),
            pltpu.VMEM((NBUF * CHUNK, K * PACK), jnp.float32),
            pltpu.VMEM((NBUF * CHUNK, K * PACK), jnp.float32),
            pltpu.VMEM((BPW,), jnp.float32),
            pltpu.SemaphoreType.DMA((NBUF,)),
        ],
    )
    return fn(w, h, uidx, iidx)


def kernel(x, W, H):
    w2 = W.reshape(W.shape[0] // PACK, K * PACK)
    h2 = H.reshape(H.shape[0] // PACK, K * PACK)
    uidx = x[:, 0].reshape(NW, NCHUNK, CHUNK)
    iidx = x[:, 1].reshape(NW, NCHUNK, CHUNK)
    return _run(w2, h2, uidx, iidx)


# speedup vs baseline: 1.0011x; 1.0011x over previous
import jax
import jax.numpy as jnp
from jax import lax
from jax.experimental import pallas as pl
from jax.experimental.pallas import tpu as pltpu
from jax.experimental.pallas import tpu_sc as plsc

B = 16384
K = 16
PACK = 8
NC = 2
NS = 16
NW = NC * NS
BPW = B // NW
NCHUNK = 4
CHUNK = BPW // NCHUNK
NBUF = 2


def _sc_kernel(w_hbm, h_hbm, uidx_hbm, iidx_hbm, out_hbm,
               uidx_v, iidx_v, usup_v, isup_v, urows_v, vrows_v, out_v, sems):
    wid = lax.axis_index("c") * NS + lax.axis_index("s")
    base = wid * BPW

    pltpu.sync_copy(uidx_hbm.at[pl.ds(base, BPW)], uidx_v)
    pltpu.sync_copy(iidx_hbm.at[pl.ds(base, BPW)], iidx_v)

    for t in range(BPW // 16):
        sl = pl.ds(t * 16, 16)
        usup_v[sl] = uidx_v[sl] >> 3
        isup_v[sl] = iidx_v[sl] >> 3

    def issue(c):
        buf = c % NBUF
        src = pl.ds(c * CHUNK, CHUNK)
        dst = pl.ds(buf * CHUNK, CHUNK)
        return (
            pltpu.async_copy(w_hbm.at[usup_v.at[src]], urows_v.at[dst], sems.at[buf]),
            pltpu.async_copy(h_hbm.at[isup_v.at[src]], vrows_v.at[dst], sems.at[buf]),
        )

    lane = lax.iota(jnp.int32, 16)
    pending = {0: issue(0), 1: issue(1)}

    for c in range(NCHUNK):
        cu, cv = pending.pop(c)
        cu.wait()
        cv.wait()
        buf = c % NBUF
        for g in range(CHUNK // 16):
            sl = pl.ds(c * CHUNK + g * 16, 16)
            rows = buf * CHUNK + g * 16 + lane
            ucol0 = (uidx_v[sl] & 7) << 4
            vcol0 = (iidx_v[sl] & 7) << 4
            acc = jnp.zeros((16,), jnp.float32)
            for k in range(K):
                u = plsc.load_gather(urows_v, [rows, ucol0 + k])
                v = plsc.load_gather(vrows_v, [rows, vcol0 + k])
                acc = acc + u * v
            out_v[pl.ds(c * CHUNK + g * 16, 16)] = acc
        if c + NBUF < NCHUNK:
            pending[c + NBUF] = issue(c + NBUF)

    pltpu.sync_copy(out_v, out_hbm.at[pl.ds(base, BPW)])


@jax.jit
def _run(w, h, uidx, iidx):
    mesh = plsc.VectorSubcoreMesh(core_axis_name="c", subcore_axis_name="s")
    fn = pl.kernel(
        _sc_kernel,
        mesh=mesh,
        compiler_params=pltpu.CompilerParams(needs_layout_passes=False),
        out_type=jax.ShapeDtypeStruct((B,), jnp.float32),
        scratch_types=[
            pltpu.VMEM((BPW,), jnp.int32),
            pltpu.VMEM((BPW,), jnp.int32),
            pltpu.VMEM((BPW,), jnp.int32),
            pltpu.VMEM((BPW,), jnp.int32),
            pltpu.VMEM((NBUF * CHUNK, K * PACK), jnp.float32),
            pltpu.VMEM((NBUF * CHUNK, K * PACK), jnp.float32),
            pltpu.VMEM((BPW,), jnp.float32),
            pltpu.SemaphoreType.DMA((NBUF,)),
        ],
    )
    return fn(w, h, uidx, iidx)


def kernel(x, W, H):
    w2 = W.reshape(W.shape[0] // PACK, K * PACK)
    h2 = H.reshape(H.shape[0] // PACK, K * PACK)
    return _run(w2, h2, x[:, 0], x[:, 1])


# speedup vs baseline: 6.1510x; 6.1445x over previous
import jax
import jax.numpy as jnp
from jax import lax
from jax.experimental import pallas as pl
from jax.experimental.pallas import tpu as pltpu
from jax.experimental.pallas import tpu_sc as plsc

B = 16384
K = 16
NC = 2
NS = 16
NW = NC * NS
BPW = B // NW
NG = BPW // 16


def _sc_kernel(wt_hbm, ht_hbm, uidx_hbm, iidx_hbm, out_hbm,
               uidx_v, iidx_v, ubuf, vbuf, out_v, sem):
    wid = lax.axis_index("c") * NS + lax.axis_index("s")
    base = wid * BPW

    pltpu.sync_copy(uidx_hbm.at[pl.ds(base, BPW)], uidx_v)
    pltpu.sync_copy(iidx_hbm.at[pl.ds(base, BPW)], iidx_v)

    lane = lax.iota(jnp.int32, 16)

    def group_body(g, _):
        sl = pl.ds(g * 16, 16)
        uvec = uidx_v[sl]
        ivec = iidx_v[sl]
        copies = []
        for e in range(16):
            bu = pl.multiple_of((uvec[e] >> 7) << 7, 128)
            bi = pl.multiple_of((ivec[e] >> 7) << 7, 128)
            for j in range(2):
                copies.append(pltpu.async_copy(
                    wt_hbm.at[j, :, pl.ds(bu, 128)],
                    ubuf.at[pl.ds((2 * e + j) * 8, 8), :], sem))
                copies.append(pltpu.async_copy(
                    ht_hbm.at[j, :, pl.ds(bi, 128)],
                    vbuf.at[pl.ds((2 * e + j) * 8, 8), :], sem))
        for cp in copies:
            cp.wait()
        acc = jnp.zeros((16,), jnp.float32)
        ucol = uvec & 127
        vcol = ivec & 127
        for k in range(K):
            rowvec = 16 * lane + k
            u = plsc.load_gather(ubuf, [rowvec, ucol])
            v = plsc.load_gather(vbuf, [rowvec, vcol])
            acc = acc + u * v
        out_v[sl] = acc
        return _

    lax.fori_loop(0, NG, group_body, None)

    pltpu.sync_copy(out_v, out_hbm.at[pl.ds(base, BPW)])


@jax.jit
def _run(wt, ht, uidx, iidx):
    mesh = plsc.VectorSubcoreMesh(core_axis_name="c", subcore_axis_name="s")
    fn = pl.kernel(
        _sc_kernel,
        mesh=mesh,
        compiler_params=pltpu.CompilerParams(needs_layout_passes=False),
        out_type=jax.ShapeDtypeStruct((B,), jnp.float32),
        scratch_types=[
            pltpu.VMEM((BPW,), jnp.int32),
            pltpu.VMEM((BPW,), jnp.int32),
            pltpu.VMEM((256, 128), jnp.float32),
            pltpu.VMEM((256, 128), jnp.float32),
            pltpu.VMEM((BPW,), jnp.float32),
            pltpu.SemaphoreType.DMA,
        ],
    )
    return fn(wt, ht, uidx, iidx)


def kernel(x, W, H):
    wt = W.T.reshape(2, 8, W.shape[0])
    ht = H.T.reshape(2, 8, H.shape[0])
    return _run(wt, ht, x[:, 0], x[:, 1])
